# baseline (speedup 1.0000x reference)
import functools

import jax
import jax.numpy as jnp
from jax import lax
from jax.experimental import pallas as pl
from jax.experimental.pallas import tpu as pltpu
from jax.experimental.pallas import tpu_sc as plsc

N = 10000
E = 320000
D = 128
ED = 16

NC = 2
NS = 16
NW = NC * NS
EPW = E // NW
CH = 120
NB = 3
NCH = 84
EPWP = NCH * CH
NOUT = NCH // NB
NP = 10240
PADROW = 10008
RPS = NP // NS

_MESH = dict(core_axis_name="c", subcore_axis_name="s")


def _scratch():
    return (
        [pltpu.VMEM_SHARED((NP, D), jnp.float32)]
        + [pltpu.VMEM((CH,), jnp.int32) for _ in range(2 * NB)]
        + [pltpu.VMEM((CH, D), jnp.float32) for _ in range(NB)]
        + [pltpu.SemaphoreType.DMA for _ in range(3)]
    )


def _sc_agg(gather: bool):
    @functools.partial(
        pl.kernel,
        out_type=jax.ShapeDtypeStruct((NC, NP, D), jnp.float32),
        mesh=plsc.VectorSubcoreMesh(**_MESH),
        scratch_types=_scratch(),
    )
    def agg(x_hbm, row_hbm, col_hbm, z128_hbm, p_out, acc_p, *bufs):
        row_v = bufs[0:NB]
        col_v = bufs[NB:2 * NB]
        msg_v = bufs[2 * NB:3 * NB]
        sem_i, sem_g, sem_s = bufs[3 * NB:]
        c = lax.axis_index("c")
        s = lax.axis_index("s")
        wid = s * NC + c
        r0 = s * RPS

        pltpu.sync_copy(z128_hbm.at[pl.ds(r0, RPS)], acc_p.at[pl.ds(r0, RPS)])
        plsc.subcore_barrier()

        e_base = wid * EPWP

        def do_batch(e00):
            ic = []
            for b in range(NB):
                e0 = e00 + b * CH
                ic.append(pltpu.async_copy(col_hbm.at[pl.ds(e0, CH)],
                                           col_v[b], sem_i))
                if gather:
                    ic.append(pltpu.async_copy(row_hbm.at[pl.ds(e0, CH)],
                                               row_v[b], sem_i))
            for d in ic:
                d.wait()
            gc = []
            for b in range(NB):
                if gather:
                    gc.append(pltpu.async_copy(x_hbm.at[row_v[b]], msg_v[b],
                                               sem_g))
                else:
                    e0 = e00 + b * CH
                    gc.append(pltpu.async_copy(x_hbm.at[pl.ds(e0, CH)],
                                               msg_v[b], sem_g))
            sc = []
            for b in range(NB):
                gc[b].wait()
                sc.append(pltpu.async_copy(msg_v[b], acc_p.at[col_v[b]],
                                           sem_s, add=True))
            for d in sc:
                d.wait()

        def body(i, carry):
            do_batch(e_base + i * (NB * CH))
            return carry

        lax.fori_loop(0, NOUT, body, 0)
        plsc.subcore_barrier()

        pltpu.sync_copy(acc_p.at[pl.ds(r0, RPS)], p_out.at[c, pl.ds(r0, RPS)])

    return agg


_BN = 1000


def _tc_dense_body(p_ref, pay_ref, h_ref, we_ref, wl_ref, wr_ref,
                   b_ref, o_ref, *, relu):
    p = p_ref[0] + p_ref[1]
    pay = pay_ref[0] + pay_ref[1]
    ea = pay[:, :ED]
    deg = pay[:, ED:ED + 1]
    inv = 1.0 / jnp.maximum(deg, 1.0)
    dn = (((1,), (1,)), ((), ()))
    e_term = lax.dot_general(ea, we_ref[...], dn,
                             preferred_element_type=jnp.float32)
    su = (p + e_term) * inv
    out = (lax.dot_general(su, wl_ref[...], dn,
                           preferred_element_type=jnp.float32)
           + lax.dot_general(h_ref[...], wr_ref[...], dn,
                             preferred_element_type=jnp.float32)
           + b_ref[...])
    if relu:
        out = jnp.maximum(out, 0.0)
    o_ref[...] = out


def _tc_dense(relu):
    return pl.pallas_call(
        functools.partial(_tc_dense_body, relu=relu),
        grid=(N // _BN,),
        in_specs=[
            pl.BlockSpec((NC, _BN, D), lambda i: (0, i, 0)),
            pl.BlockSpec((NC, _BN, D), lambda i: (0, i, 0)),
            pl.BlockSpec((_BN, D), lambda i: (i, 0)),
            pl.BlockSpec((D, ED), lambda i: (0, 0)),
            pl.BlockSpec((D, D), lambda i: (0, 0)),
            pl.BlockSpec((D, D), lambda i: (0, 0)),
            pl.BlockSpec((1, D), lambda i: (0, 0)),
        ],
        out_specs=pl.BlockSpec((_BN, D), lambda i: (i, 0)),
        out_shape=jax.ShapeDtypeStruct((N, D), jnp.float32),
    )


def kernel(x, edge_index, edge_attr, W_edge0, W_l0, b_l0, W_r0,
           W_edge1, W_l1, b_l1, W_r1):
    npad = EPWP - EPW
    row1 = jnp.concatenate(
        [edge_index[0].reshape(NW, EPW),
         jnp.zeros((NW, npad), jnp.int32)], axis=1).reshape(-1)
    col1 = jnp.concatenate(
        [edge_index[1].reshape(NW, EPW),
         jnp.full((NW, npad), PADROW, jnp.int32)], axis=1).reshape(-1)
    pay = jnp.concatenate(
        [edge_attr, jnp.ones((E, 1), jnp.float32),
         jnp.zeros((E, D - ED - 1), jnp.float32)], axis=1)
    payp = jnp.concatenate(
        [pay.reshape(NW, EPW, D),
         jnp.zeros((NW, npad, D), jnp.float32)], axis=1).reshape(-1, D)
    z128 = jnp.zeros((NP, D), jnp.float32)

    pp = _sc_agg(False)(payp, col1, col1, z128)
    p0 = _sc_agg(True)(x, row1, col1, z128)
    h1 = _tc_dense(True)(p0, pp, x, W_edge0, W_l0, W_r0, b_l0.reshape(1, D))
    p1 = _sc_agg(True)(h1, row1, col1, z128)
    out = _tc_dense(False)(p1, pp, h1, W_edge1, W_l1, W_r1,
                           b_l1.reshape(1, D))
    return out


# speedup vs baseline: 1.5153x; 1.5153x over previous
import functools

import jax
import jax.numpy as jnp
from jax import lax
from jax.experimental import pallas as pl
from jax.experimental.pallas import tpu as pltpu
from jax.experimental.pallas import tpu_sc as plsc

N = 10000
E = 320000
D = 128
ED = 16

NC = 2
NS = 16
NW = NC * NS
EPW = E // NW
CH = 80
NB = 4
NCH = EPW // CH
NOUT = NCH // NB
NREM = NCH - NOUT * NB
NP = 10240
RPS = NP // NS

_MESH = dict(core_axis_name="c", subcore_axis_name="s")


def _scratch():
    return (
        [pltpu.VMEM_SHARED((NP, D), jnp.float32)]
        + [pltpu.VMEM((CH,), jnp.int32) for _ in range(2 * NB)]
        + [pltpu.VMEM((CH, D), jnp.float32) for _ in range(NB)]
        + [pltpu.SemaphoreType.DMA for _ in range(3)]
    )


def _sc_agg(gather: bool):
    @functools.partial(
        pl.kernel,
        out_type=jax.ShapeDtypeStruct((NC, NP, D), jnp.float32),
        mesh=plsc.VectorSubcoreMesh(**_MESH),
        scratch_types=_scratch(),
    )
    def agg(x_hbm, row_hbm, col_hbm, z128_hbm, p_out, acc_p, *bufs):
        row_v = bufs[0:NB]
        col_v = bufs[NB:2 * NB]
        msg_v = bufs[2 * NB:3 * NB]
        sem_i, sem_g, sem_s = bufs[3 * NB:]
        c = lax.axis_index("c")
        s = lax.axis_index("s")
        wid = s * NC + c
        r0 = s * RPS

        pltpu.sync_copy(z128_hbm.at[pl.ds(r0, RPS)], acc_p.at[pl.ds(r0, RPS)])
        plsc.subcore_barrier()

        e_base = wid * EPW

        def do_batch(e00, nb):
            ic = []
            for b in range(nb):
                e0 = e00 + b * CH
                ic.append(pltpu.async_copy(col_hbm.at[pl.ds(e0, CH)],
                                           col_v[b], sem_i))
                if gather:
                    ic.append(pltpu.async_copy(row_hbm.at[pl.ds(e0, CH)],
                                               row_v[b], sem_i))
            for d in ic:
                d.wait()
            gc = []
            for b in range(nb):
                if gather:
                    gc.append(pltpu.async_copy(x_hbm.at[row_v[b]], msg_v[b],
                                               sem_g))
                else:
                    e0 = e00 + b * CH
                    gc.append(pltpu.async_copy(x_hbm.at[pl.ds(e0, CH)],
                                               msg_v[b], sem_g))
            sc = []
            for b in range(nb):
                gc[b].wait()
                sc.append(pltpu.async_copy(msg_v[b], acc_p.at[col_v[b]],
                                           sem_s, add=True))
            for d in sc:
                d.wait()

        def body(i, carry):
            do_batch(e_base + i * (NB * CH), NB)
            return carry

        lax.fori_loop(0, NOUT, body, 0)
        do_batch(e_base + NOUT * NB * CH, NREM)
        plsc.subcore_barrier()

        pltpu.sync_copy(acc_p.at[pl.ds(r0, RPS)], p_out.at[c, pl.ds(r0, RPS)])

    return agg


_BN = 1000


def _tc_dense_body(p_ref, pay_ref, h_ref, we_ref, wl_ref, wr_ref,
                   b_ref, o_ref, *, relu):
    p = p_ref[0] + p_ref[1]
    pay = pay_ref[0] + pay_ref[1]
    ea = pay[:, :ED]
    deg = pay[:, ED:ED + 1]
    inv = 1.0 / jnp.maximum(deg, 1.0)
    dn = (((1,), (1,)), ((), ()))
    e_term = lax.dot_general(ea, we_ref[...], dn,
                             preferred_element_type=jnp.float32)
    su = (p + e_term) * inv
    out = (lax.dot_general(su, wl_ref[...], dn,
                           preferred_element_type=jnp.float32)
           + lax.dot_general(h_ref[...], wr_ref[...], dn,
                             preferred_element_type=jnp.float32)
           + b_ref[...])
    if relu:
        out = jnp.maximum(out, 0.0)
    o_ref[...] = out


def _tc_dense(relu):
    return pl.pallas_call(
        functools.partial(_tc_dense_body, relu=relu),
        grid=(N // _BN,),
        in_specs=[
            pl.BlockSpec((NC, _BN, D), lambda i: (0, i, 0)),
            pl.BlockSpec((NC, _BN, D), lambda i: (0, i, 0)),
            pl.BlockSpec((_BN, D), lambda i: (i, 0)),
            pl.BlockSpec((D, ED), lambda i: (0, 0)),
            pl.BlockSpec((D, D), lambda i: (0, 0)),
            pl.BlockSpec((D, D), lambda i: (0, 0)),
            pl.BlockSpec((1, D), lambda i: (0, 0)),
        ],
        out_specs=pl.BlockSpec((_BN, D), lambda i: (i, 0)),
        out_shape=jax.ShapeDtypeStruct((N, D), jnp.float32),
    )


def kernel(x, edge_index, edge_attr, W_edge0, W_l0, b_l0, W_r0,
           W_edge1, W_l1, b_l1, W_r1):
    row1 = edge_index[0]
    col1 = edge_index[1]
    pay = jnp.concatenate(
        [edge_attr, jnp.ones((E, 1), jnp.float32),
         jnp.zeros((E, D - ED - 1), jnp.float32)], axis=1)
    z128 = jnp.zeros((NP, D), jnp.float32)

    pp = _sc_agg(False)(pay, col1, col1, z128)
    p0 = _sc_agg(True)(x, row1, col1, z128)
    h1 = _tc_dense(True)(p0, pp, x, W_edge0, W_l0, W_r0, b_l0.reshape(1, D))
    p1 = _sc_agg(True)(h1, row1, col1, z128)
    out = _tc_dense(False)(p1, pp, h1, W_edge1, W_l1, W_r1,
                           b_l1.reshape(1, D))
    return out


# speedup vs baseline: 1.5706x; 1.0365x over previous
import functools

import jax
import jax.numpy as jnp
from jax import lax
from jax.experimental import pallas as pl
from jax.experimental.pallas import tpu as pltpu
from jax.experimental.pallas import tpu_sc as plsc

N = 10000
E = 320000
D = 128
ED = 16

NC = 2
NS = 16
NW = NC * NS
EPW = E // NW
CH = 80
NB = 4
NCH = EPW // CH
NOUT = NCH // NB
NREM = NCH - NOUT * NB
NP = 10240
RPS = NP // NS

_MESH = dict(core_axis_name="c", subcore_axis_name="s")


def _scratch():
    return (
        [pltpu.VMEM_SHARED((NP, D), jnp.float32)]
        + [pltpu.VMEM((CH,), jnp.int32) for _ in range(2 * NB)]
        + [pltpu.VMEM((CH, D), jnp.float32) for _ in range(NB)]
        + [pltpu.SemaphoreType.DMA for _ in range(3)]
    )


def _sc_agg(gather: bool):
    @functools.partial(
        pl.kernel,
        out_type=jax.ShapeDtypeStruct((NC, NP, D), jnp.float32),
        mesh=plsc.VectorSubcoreMesh(**_MESH),
        scratch_types=_scratch(),
    )
    def agg(x_hbm, row_hbm, col_hbm, z128_hbm, p_out, acc_p, *bufs):
        row_v = bufs[0:NB]
        col_v = bufs[NB:2 * NB]
        msg_v = bufs[2 * NB:3 * NB]
        sem_i, sem_g, sem_s = bufs[3 * NB:]
        c = lax.axis_index("c")
        s = lax.axis_index("s")
        wid = s * NC + c
        r0 = s * RPS

        pltpu.sync_copy(z128_hbm.at[pl.ds(r0, RPS)], acc_p.at[pl.ds(r0, RPS)])
        plsc.subcore_barrier()

        e_base = wid * EPW

        def do_batch(e00, nb):
            ic = []
            gc = [None] * nb
            for b in range(nb):
                e0 = e00 + b * CH
                ic.append([pltpu.async_copy(col_hbm.at[pl.ds(e0, CH)],
                                            col_v[b], sem_i)])
                if gather:
                    ic[b].append(pltpu.async_copy(row_hbm.at[pl.ds(e0, CH)],
                                                  row_v[b], sem_i))
                else:
                    gc[b] = pltpu.async_copy(x_hbm.at[pl.ds(e0, CH)],
                                             msg_v[b], sem_g)
            if gather:
                for b in range(nb):
                    for d in ic[b]:
                        d.wait()
                    gc[b] = pltpu.async_copy(x_hbm.at[row_v[b]], msg_v[b],
                                             sem_g)
            sc = []
            for b in range(nb):
                if not gather:
                    for d in ic[b]:
                        d.wait()
                gc[b].wait()
                sc.append(pltpu.async_copy(msg_v[b], acc_p.at[col_v[b]],
                                           sem_s, add=True))
            for d in sc:
                d.wait()

        def body(i, carry):
            do_batch(e_base + i * (NB * CH), NB)
            return carry

        lax.fori_loop(0, NOUT, body, 0)
        do_batch(e_base + NOUT * NB * CH, NREM)
        plsc.subcore_barrier()

        pltpu.sync_copy(acc_p.at[pl.ds(r0, RPS)], p_out.at[c, pl.ds(r0, RPS)])

    return agg


_BN = 1000


def _tc_dense_body(p_ref, pay_ref, h_ref, we_ref, wl_ref, wr_ref,
                   b_ref, o_ref, *, relu):
    p = p_ref[0] + p_ref[1]
    pay = pay_ref[0] + pay_ref[1]
    ea = pay[:, :ED]
    deg = pay[:, ED:ED + 1]
    inv = 1.0 / jnp.maximum(deg, 1.0)
    dn = (((1,), (1,)), ((), ()))
    e_term = lax.dot_general(ea, we_ref[...], dn,
                             preferred_element_type=jnp.float32)
    su = (p + e_term) * inv
    out = (lax.dot_general(su, wl_ref[...], dn,
                           preferred_element_type=jnp.float32)
           + lax.dot_general(h_ref[...], wr_ref[...], dn,
                             preferred_element_type=jnp.float32)
           + b_ref[...])
    if relu:
        out = jnp.maximum(out, 0.0)
    o_ref[...] = out


def _tc_dense(relu):
    return pl.pallas_call(
        functools.partial(_tc_dense_body, relu=relu),
        grid=(N // _BN,),
        in_specs=[
            pl.BlockSpec((NC, _BN, D), lambda i: (0, i, 0)),
            pl.BlockSpec((NC, _BN, D), lambda i: (0, i, 0)),
            pl.BlockSpec((_BN, D), lambda i: (i, 0)),
            pl.BlockSpec((D, ED), lambda i: (0, 0)),
            pl.BlockSpec((D, D), lambda i: (0, 0)),
            pl.BlockSpec((D, D), lambda i: (0, 0)),
            pl.BlockSpec((1, D), lambda i: (0, 0)),
        ],
        out_specs=pl.BlockSpec((_BN, D), lambda i: (i, 0)),
        out_shape=jax.ShapeDtypeStruct((N, D), jnp.float32),
    )


def kernel(x, edge_index, edge_attr, W_edge0, W_l0, b_l0, W_r0,
           W_edge1, W_l1, b_l1, W_r1):
    row1 = edge_index[0]
    col1 = edge_index[1]
    pay = jnp.concatenate(
        [edge_attr, jnp.ones((E, 1), jnp.float32),
         jnp.zeros((E, D - ED - 1), jnp.float32)], axis=1)
    z128 = jnp.zeros((NP, D), jnp.float32)

    pp = _sc_agg(False)(pay, col1, col1, z128)
    p0 = _sc_agg(True)(x, row1, col1, z128)
    h1 = _tc_dense(True)(p0, pp, x, W_edge0, W_l0, W_r0, b_l0.reshape(1, D))
    p1 = _sc_agg(True)(h1, row1, col1, z128)
    out = _tc_dense(False)(p1, pp, h1, W_edge1, W_l1, W_r1,
                           b_l1.reshape(1, D))
    return out
